# initial kernel scaffold (unmeasured)
import jax
import jax.numpy as jnp
from jax import lax
from jax.experimental import pallas as pl
from jax.experimental.pallas import tpu as pltpu

N_DEV = 8


def kernel(x, w_mat):
    m, k_per = x.shape
    n = w_mat.shape[1]
    chunk = m // N_DEV
    f32 = jnp.float32
    f8 = jnp.float8_e4m3fn

    def body(x_ref, w_ref, out_ref,
             comm_ref, qcomm_ref, stag_ref, amax_ref,
             rs_send, rs_recv, ag_send, ag_recv,
             sc_send, sc_recv, out_sem,
             credit_rs, credit_ag):
        p = lax.axis_index("i")
        left = lax.rem(p + N_DEV - 1, N_DEV)
        right = lax.rem(p + 1, N_DEV)

        barrier = pltpu.get_barrier_semaphore()
        for nbr in (left, right):
            pl.semaphore_signal(barrier, inc=1, device_id=(nbr,),
                                device_id_type=pl.DeviceIdType.MESH)
        pl.semaphore_wait(barrier, 2)

        def partial_chunk(c):
            xs = x_ref[pl.ds(c * chunk, chunk), :]
            return jnp.dot(xs, w_ref[...], preferred_element_type=f32)

        comm_ref[0] = partial_chunk(lax.rem(p + N_DEV - 1, N_DEV))
        for t in range(N_DEV - 1):
            s = t % 2
            r = (t + 1) % 2
            if t >= 1:
                pl.semaphore_wait(credit_rs, 1)
            rdma = pltpu.make_async_remote_copy(
                src_ref=comm_ref.at[s], dst_ref=comm_ref.at[r],
                send_sem=rs_send.at[s], recv_sem=rs_recv.at[r],
                device_id=(right,), device_id_type=pl.DeviceIdType.MESH)
            rdma.start()
            rdma.wait()
            if t <= N_DEV - 3:
                pl.semaphore_signal(credit_rs, inc=1, device_id=(left,),
                                    device_id_type=pl.DeviceIdType.MESH)
            c = lax.rem(p + 2 * N_DEV - 2 - t, N_DEV)
            if t < N_DEV - 2:
                comm_ref[r] = comm_ref[r] + partial_chunk(c)
            else:
                comm_ref[r] = jnp.maximum(comm_ref[r] + partial_chunk(c), 0.0)

        rf = (N_DEV - 1) % 2

        local_amax = jnp.max(comm_ref[rf])
        amax_ref[pl.ds(p, 1), :] = jnp.full((1, 128), local_amax, f32)
        for h in range(N_DEV - 1):
            row_s = lax.rem(p + N_DEV - h, N_DEV)
            row_r = lax.rem(p + N_DEV - 1 - h, N_DEV)
            sc = pltpu.make_async_remote_copy(
                src_ref=amax_ref.at[pl.ds(row_s, 1), :],
                dst_ref=amax_ref.at[pl.ds(row_s, 1), :],
                send_sem=sc_send.at[h], recv_sem=sc_recv.at[h],
                device_id=(right,), device_id_type=pl.DeviceIdType.MESH)
            sc.start()
            sc.wait()
            del row_r
        amax_g = jnp.max(amax_ref[...])
        scale = amax_g / 448.0
        inv = jnp.where(amax_g > 0.0, 448.0 / amax_g, 0.0)

        qcomm_ref[0] = (comm_ref[rf] * inv).astype(f8)
        stag_ref[...] = qcomm_ref[0].astype(f32) * scale
        cp = pltpu.make_async_copy(
            stag_ref, out_ref.at[pl.ds(p * chunk, chunk), :], out_sem)
        cp.start()
        prev_cp = cp
        for h in range(N_DEV - 1):
            s = h % 2
            r = (h + 1) % 2
            if h >= 1:
                pl.semaphore_wait(credit_ag, 1)
            rdma = pltpu.make_async_remote_copy(
                src_ref=qcomm_ref.at[s], dst_ref=qcomm_ref.at[r],
                send_sem=ag_send.at[s], recv_sem=ag_recv.at[r],
                device_id=(right,), device_id_type=pl.DeviceIdType.MESH)
            rdma.start()
            rdma.wait()
            if h <= N_DEV - 3:
                pl.semaphore_signal(credit_ag, inc=1, device_id=(left,),
                                    device_id_type=pl.DeviceIdType.MESH)
            prev_cp.wait()
            origin = lax.rem(p + N_DEV - 1 - h, N_DEV)
            stag_ref[...] = qcomm_ref[r].astype(f32) * scale
            cp = pltpu.make_async_copy(
                stag_ref, out_ref.at[pl.ds(origin * chunk, chunk), :], out_sem)
            cp.start()
            prev_cp = cp
        prev_cp.wait()

    return pl.pallas_call(
        body,
        out_shape=jax.ShapeDtypeStruct((m, n), f32),
        in_specs=[pl.BlockSpec(memory_space=pltpu.VMEM),
                  pl.BlockSpec(memory_space=pltpu.VMEM)],
        out_specs=pl.BlockSpec(memory_space=pltpu.ANY),
        scratch_shapes=[
            pltpu.VMEM((2, chunk, n), f32),
            pltpu.VMEM((2, chunk, n), f8),
            pltpu.VMEM((chunk, n), f32),
            pltpu.VMEM((N_DEV, 128), f32),
            pltpu.SemaphoreType.DMA((2,)),
            pltpu.SemaphoreType.DMA((2,)),
            pltpu.SemaphoreType.DMA((2,)),
            pltpu.SemaphoreType.DMA((2,)),
            pltpu.SemaphoreType.DMA((N_DEV - 1,)),
            pltpu.SemaphoreType.DMA((N_DEV - 1,)),
            pltpu.SemaphoreType.DMA,
            pltpu.SemaphoreType.REGULAR,
            pltpu.SemaphoreType.REGULAR,
        ],
        compiler_params=pltpu.CompilerParams(collective_id=0),
    )(x, w_mat)


# baseline (device time: 1818471 ns/iter reference)
import jax
import jax.numpy as jnp
from jax import lax
from jax.experimental import pallas as pl
from jax.experimental.pallas import tpu as pltpu

N_DEV = 8
N_HALF = 2


def kernel(x, w_mat):
    m, k_per = x.shape
    n = w_mat.shape[1]
    chunk = m // N_DEV
    ncol = n // N_HALF
    f32 = jnp.float32
    f8 = jnp.float8_e4m3fn

    def body(x_ref, w_ref, out_ref,
             comm_ref, qcomm_ref, amax_ref,
             rs_send, rs_recv, ag_send, ag_recv,
             sc_send, sc_recv, out_sem,
             credit_rs, credit_ag):
        p = lax.axis_index("i")
        left = lax.rem(p + N_DEV - 1, N_DEV)
        right = lax.rem(p + 1, N_DEV)
        my_rows = pl.ds(p * chunk, chunk)

        def signal(sem, target):
            pl.semaphore_signal(sem, inc=1, device_id=(target,),
                                device_id_type=pl.DeviceIdType.MESH)

        barrier = pltpu.get_barrier_semaphore()
        signal(barrier, left)
        signal(barrier, right)
        pl.semaphore_wait(barrier, 2)

        STRIP = 2048
        n_strip = ncol // STRIP

        def accum_partial(r, c, col0, relu):
            xs = x_ref[pl.ds(c * chunk, chunk), :]
            for j in range(n_strip):
                sl = pl.ds(j * STRIP, STRIP)
                wsl = pl.ds(col0 + j * STRIP, STRIP)
                acc = comm_ref[r, :, sl] + jnp.dot(
                    xs, w_ref[:, wsl], preferred_element_type=f32)
                if relu:
                    acc = jnp.maximum(acc, 0.0)
                comm_ref[r, :, sl] = acc

        def init_partial(c, col0):
            xs = x_ref[pl.ds(c * chunk, chunk), :]
            for j in range(n_strip):
                sl = pl.ds(j * STRIP, STRIP)
                wsl = pl.ds(col0 + j * STRIP, STRIP)
                comm_ref[0, :, sl] = jnp.dot(
                    xs, w_ref[:, wsl], preferred_element_type=f32)

        local_amax = jnp.float32(0.0)
        for half in range(N_HALF):
            col0 = half * ncol
            cols = pl.ds(col0, ncol)
            init_partial(lax.rem(p + N_DEV - 1, N_DEV), col0)
            for t in range(N_DEV - 1):
                s = t % 2
                r = (t + 1) % 2
                if not (half == 0 and t == 0):
                    pl.semaphore_wait(credit_rs, 1)
                rdma = pltpu.make_async_remote_copy(
                    src_ref=comm_ref.at[s], dst_ref=comm_ref.at[r],
                    send_sem=rs_send.at[s], recv_sem=rs_recv.at[r],
                    device_id=(right,), device_id_type=pl.DeviceIdType.MESH)
                rdma.start()
                rdma.wait()
                if t <= N_DEV - 3:
                    signal(credit_rs, left)
                c = lax.rem(p + 2 * N_DEV - 2 - t, N_DEV)
                accum_partial(r, c, col0, relu=(t == N_DEV - 2))
            rf = (N_DEV - 1) % 2
            local_amax = jnp.maximum(local_amax, jnp.max(comm_ref[rf]))
            park = pltpu.make_async_copy(
                comm_ref.at[rf], out_ref.at[my_rows, cols], out_sem)
            park.start()
            park.wait()
            if half == 0:
                signal(credit_rs, left)

        amax_ref[pl.ds(p, 1), :] = jnp.full((1, 128), local_amax, f32)
        for h in range(N_DEV - 1):
            row_s = lax.rem(p + N_DEV - h, N_DEV)
            sc = pltpu.make_async_remote_copy(
                src_ref=amax_ref.at[pl.ds(row_s, 1), :],
                dst_ref=amax_ref.at[pl.ds(row_s, 1), :],
                send_sem=sc_send.at[h], recv_sem=sc_recv.at[h],
                device_id=(right,), device_id_type=pl.DeviceIdType.MESH)
            sc.start()
            sc.wait()
        amax_g = jnp.max(amax_ref[...])
        scale = amax_g / 448.0
        inv = jnp.where(amax_g > 0.0, 448.0 / amax_g, 0.0)

        for half in range(N_HALF):
            col0 = half * ncol
            cols = pl.ds(col0, ncol)
            rb = pltpu.make_async_copy(
                out_ref.at[my_rows, cols], comm_ref.at[0], out_sem)
            rb.start()
            rb.wait()
            for j in range(n_strip):
                sl = pl.ds(j * STRIP, STRIP)
                qcomm_ref[0, :, sl] = (comm_ref[0, :, sl] * inv).astype(f8)
                comm_ref[1, :, sl] = qcomm_ref[0, :, sl].astype(f32) * scale
            cp = pltpu.make_async_copy(
                comm_ref.at[1], out_ref.at[my_rows, cols], out_sem)
            cp.start()
            prev_cp = cp
            for h in range(N_DEV - 1):
                s = h % 2
                r = (h + 1) % 2
                if not (half == 0 and h == 0):
                    pl.semaphore_wait(credit_ag, 1)
                rdma = pltpu.make_async_remote_copy(
                    src_ref=qcomm_ref.at[s], dst_ref=qcomm_ref.at[r],
                    send_sem=ag_send.at[s], recv_sem=ag_recv.at[r],
                    device_id=(right,), device_id_type=pl.DeviceIdType.MESH)
                rdma.start()
                rdma.wait()
                if h <= N_DEV - 3:
                    signal(credit_ag, left)
                prev_cp.wait()
                origin = lax.rem(p + N_DEV - 1 - h, N_DEV)
                for j in range(n_strip):
                    sl = pl.ds(j * STRIP, STRIP)
                    comm_ref[1, :, sl] = qcomm_ref[r, :, sl].astype(f32) * scale
                cp = pltpu.make_async_copy(
                    comm_ref.at[1],
                    out_ref.at[pl.ds(origin * chunk, chunk), cols], out_sem)
                cp.start()
                prev_cp = cp
            prev_cp.wait()
            if half == 0:
                signal(credit_ag, left)

    return pl.pallas_call(
        body,
        out_shape=jax.ShapeDtypeStruct((m, n), f32),
        in_specs=[pl.BlockSpec(memory_space=pltpu.VMEM),
                  pl.BlockSpec(memory_space=pltpu.VMEM)],
        out_specs=pl.BlockSpec(memory_space=pl.ANY),
        scratch_shapes=[
            pltpu.VMEM((2, chunk, ncol), f32),
            pltpu.VMEM((2, chunk, ncol), f8),
            pltpu.VMEM((N_DEV, 128), f32),
            pltpu.SemaphoreType.DMA((2,)),
            pltpu.SemaphoreType.DMA((2,)),
            pltpu.SemaphoreType.DMA((2,)),
            pltpu.SemaphoreType.DMA((2,)),
            pltpu.SemaphoreType.DMA((N_DEV - 1,)),
            pltpu.SemaphoreType.DMA((N_DEV - 1,)),
            pltpu.SemaphoreType.DMA,
            pltpu.SemaphoreType.REGULAR,
            pltpu.SemaphoreType.REGULAR,
        ],
        compiler_params=pltpu.CompilerParams(
            collective_id=0, vmem_limit_bytes=40 * 1024 * 1024),
    )(x, w_mat)


# device time: 1803558 ns/iter; 1.0083x vs baseline; 1.0083x over previous
import jax
import jax.numpy as jnp
from jax import lax
from jax.experimental import pallas as pl
from jax.experimental.pallas import tpu as pltpu

N_DEV = 8
N_HALF = 2


def kernel(x, w_mat):
    m, k_per = x.shape
    n = w_mat.shape[1]
    chunk = m // N_DEV
    ncol = n // N_HALF
    f32 = jnp.float32
    f8 = jnp.float8_e4m3fn

    def body(x_ref, w_ref, out_ref,
             comm_ref, qcomm_ref, pbuf_ref, xbuf_ref, amax_ref,
             rs_send, rs_recv, ag_send, ag_recv,
             sc_send, sc_recv, out_sem, x_sem,
             credit_rs, credit_ag):
        p = lax.axis_index("i")
        left = lax.rem(p + N_DEV - 1, N_DEV)
        right = lax.rem(p + 1, N_DEV)
        my_rows = pl.ds(p * chunk, chunk)

        def signal(sem, target):
            pl.semaphore_signal(sem, inc=1, device_id=(target,),
                                device_id_type=pl.DeviceIdType.MESH)

        barrier = pltpu.get_barrier_semaphore()
        signal(barrier, left)
        signal(barrier, right)
        pl.semaphore_wait(barrier, 2)

        STRIP = 2048
        n_strip = ncol // STRIP

        def load_x(c):
            cp = pltpu.make_async_copy(
                x_ref.at[pl.ds(c * chunk, chunk), :], xbuf_ref, x_sem)
            cp.start()
            cp.wait()

        def compute_partial(c, col0):
            load_x(c)
            for j in range(n_strip):
                wsl = pl.ds(col0 + j * STRIP, STRIP)
                pbuf_ref[:, pl.ds(j * STRIP, STRIP)] = jnp.dot(
                    xbuf_ref[...], w_ref[:, wsl], preferred_element_type=f32)

        def accum_partial(r, relu):
            for j in range(n_strip):
                sl = pl.ds(j * STRIP, STRIP)
                acc = comm_ref[r, :, sl] + pbuf_ref[:, sl]
                if relu:
                    acc = jnp.maximum(acc, 0.0)
                comm_ref[r, :, sl] = acc

        local_amax = jnp.float32(0.0)
        for half in range(N_HALF):
            col0 = half * ncol
            cols = pl.ds(col0, ncol)
            compute_partial(lax.rem(p + N_DEV - 1, N_DEV), col0)
            for j in range(n_strip):
                sl = pl.ds(j * STRIP, STRIP)
                comm_ref[0, :, sl] = pbuf_ref[:, sl]
            for t in range(N_DEV - 1):
                s = t % 2
                r = (t + 1) % 2
                if not (half == 0 and t == 0):
                    pl.semaphore_wait(credit_rs, 1)
                rdma = pltpu.make_async_remote_copy(
                    src_ref=comm_ref.at[s], dst_ref=comm_ref.at[r],
                    send_sem=rs_send.at[s], recv_sem=rs_recv.at[r],
                    device_id=(right,), device_id_type=pl.DeviceIdType.MESH)
                rdma.start()
                compute_partial(lax.rem(p + 2 * N_DEV - 2 - t, N_DEV), col0)
                rdma.wait()
                if t <= N_DEV - 3:
                    signal(credit_rs, left)
                accum_partial(r, relu=(t == N_DEV - 2))
            rf = (N_DEV - 1) % 2
            local_amax = jnp.maximum(local_amax, jnp.max(comm_ref[rf]))
            park = pltpu.make_async_copy(
                comm_ref.at[rf], out_ref.at[my_rows, cols], out_sem)
            park.start()
            park.wait()
            if half == 0:
                signal(credit_rs, left)

        amax_ref[pl.ds(p, 1), :] = jnp.full((1, 128), local_amax, f32)
        for h in range(N_DEV - 1):
            row_s = lax.rem(p + N_DEV - h, N_DEV)
            sc = pltpu.make_async_remote_copy(
                src_ref=amax_ref.at[pl.ds(row_s, 1), :],
                dst_ref=amax_ref.at[pl.ds(row_s, 1), :],
                send_sem=sc_send.at[h], recv_sem=sc_recv.at[h],
                device_id=(right,), device_id_type=pl.DeviceIdType.MESH)
            sc.start()
            sc.wait()
        amax_g = jnp.max(amax_ref[...])
        scale = amax_g / 448.0
        inv = jnp.where(amax_g > 0.0, 448.0 / amax_g, 0.0)

        for half in range(N_HALF):
            col0 = half * ncol
            cols = pl.ds(col0, ncol)
            rb = pltpu.make_async_copy(
                out_ref.at[my_rows, cols], comm_ref.at[0], out_sem)
            rb.start()
            rb.wait()
            for j in range(n_strip):
                sl = pl.ds(j * STRIP, STRIP)
                qcomm_ref[0, :, sl] = (comm_ref[0, :, sl] * inv).astype(f8)
            pend_slot, pend_origin = 0, p
            prev_cp = None
            for h in range(N_DEV):
                if h < N_DEV - 1:
                    if not (half == 0 and h == 0):
                        pl.semaphore_wait(credit_ag, 1)
                    rdma = pltpu.make_async_remote_copy(
                        src_ref=qcomm_ref.at[h % 2],
                        dst_ref=qcomm_ref.at[(h + 1) % 2],
                        send_sem=ag_send.at[h % 2],
                        recv_sem=ag_recv.at[(h + 1) % 2],
                        device_id=(right,),
                        device_id_type=pl.DeviceIdType.MESH)
                    rdma.start()
                if prev_cp is not None:
                    prev_cp.wait()
                for j in range(n_strip):
                    sl = pl.ds(j * STRIP, STRIP)
                    comm_ref[1, :, sl] = (
                        qcomm_ref[pend_slot, :, sl].astype(f32) * scale)
                cp = pltpu.make_async_copy(
                    comm_ref.at[1],
                    out_ref.at[pl.ds(pend_origin * chunk, chunk), cols],
                    out_sem)
                cp.start()
                prev_cp = cp
                if h < N_DEV - 1:
                    rdma.wait()
                    if h <= N_DEV - 3:
                        signal(credit_ag, left)
                    pend_slot = (h + 1) % 2
                    pend_origin = lax.rem(p + N_DEV - 1 - h, N_DEV)
            prev_cp.wait()
            if half == 0:
                signal(credit_ag, left)

    return pl.pallas_call(
        body,
        out_shape=jax.ShapeDtypeStruct((m, n), f32),
        in_specs=[pl.BlockSpec(memory_space=pl.ANY),
                  pl.BlockSpec(memory_space=pltpu.VMEM)],
        out_specs=pl.BlockSpec(memory_space=pl.ANY),
        scratch_shapes=[
            pltpu.VMEM((2, chunk, ncol), f32),
            pltpu.VMEM((2, chunk, ncol), f8),
            pltpu.VMEM((chunk, ncol), f32),
            pltpu.VMEM((chunk, k_per), f32),
            pltpu.VMEM((N_DEV, 128), f32),
            pltpu.SemaphoreType.DMA((2,)),
            pltpu.SemaphoreType.DMA((2,)),
            pltpu.SemaphoreType.DMA((2,)),
            pltpu.SemaphoreType.DMA((2,)),
            pltpu.SemaphoreType.DMA((N_DEV - 1,)),
            pltpu.SemaphoreType.DMA((N_DEV - 1,)),
            pltpu.SemaphoreType.DMA,
            pltpu.SemaphoreType.DMA,
            pltpu.SemaphoreType.REGULAR,
            pltpu.SemaphoreType.REGULAR,
        ],
        compiler_params=pltpu.CompilerParams(
            collective_id=0, vmem_limit_bytes=46 * 1024 * 1024),
    )(x, w_mat)


# device time: 1021841 ns/iter; 1.7796x vs baseline; 1.7650x over previous
import jax
import jax.numpy as jnp
from jax import lax
from jax.experimental import pallas as pl
from jax.experimental.pallas import tpu as pltpu

N_DEV = 8
N_HALF = 2


def kernel(x, w_mat):
    m, k_per = x.shape
    n = w_mat.shape[1]
    chunk = m // N_DEV
    ncol = n // N_HALF
    hcol = ncol // 2
    f32 = jnp.float32
    f8 = jnp.float8_e4m3fn

    def body(x_ref, w_ref, out_ref,
             comm_ref, qcomm_ref, pbuf_ref, xbuf_ref, amax_ref,
             rs_send_cw, rs_recv_cw, rs_send_ccw, rs_recv_ccw,
             ag_send_cw, ag_recv_cw, ag_send_ccw, ag_recv_ccw,
             sc_send, sc_recv, out_lo_sem, out_hi_sem, x_sem,
             credit_rs_cw, credit_rs_ccw, credit_ag_cw, credit_ag_ccw):
        p = lax.axis_index("i")
        left = lax.rem(p + N_DEV - 1, N_DEV)
        right = lax.rem(p + 1, N_DEV)
        my_rows = pl.ds(p * chunk, chunk)
        LO = pl.ds(0, hcol)
        HI = pl.ds(hcol, hcol)

        def signal(sem, target):
            pl.semaphore_signal(sem, inc=1, device_id=(target,),
                                device_id_type=pl.DeviceIdType.MESH)

        barrier = pltpu.get_barrier_semaphore()
        signal(barrier, left)
        signal(barrier, right)
        pl.semaphore_wait(barrier, 2)

        STRIP = 2048
        n_strip = ncol // STRIP

        def load_x(c):
            cp = pltpu.make_async_copy(
                x_ref.at[pl.ds(c * chunk, chunk), :], xbuf_ref, x_sem)
            cp.start()
            cp.wait()

        def partial_into(dst_cols, c, wcol0):
            load_x(c)
            pbuf_ref[:, dst_cols] = jnp.dot(
                xbuf_ref[...], w_ref[:, pl.ds(wcol0, hcol)],
                preferred_element_type=f32)

        def accum_partial(r, relu):
            for j in range(n_strip):
                sl = pl.ds(j * STRIP, STRIP)
                acc = comm_ref[r, :, sl] + pbuf_ref[:, sl]
                if relu:
                    acc = jnp.maximum(acc, 0.0)
                comm_ref[r, :, sl] = acc

        def ring_pair(buf_ref, s, r, send_cw, recv_cw, send_ccw, recv_ccw):
            cw = pltpu.make_async_remote_copy(
                src_ref=buf_ref.at[s, :, LO], dst_ref=buf_ref.at[r, :, LO],
                send_sem=send_cw.at[s], recv_sem=recv_cw.at[r],
                device_id=(right,), device_id_type=pl.DeviceIdType.MESH)
            ccw = pltpu.make_async_remote_copy(
                src_ref=buf_ref.at[s, :, HI], dst_ref=buf_ref.at[r, :, HI],
                send_sem=send_ccw.at[s], recv_sem=recv_ccw.at[r],
                device_id=(left,), device_id_type=pl.DeviceIdType.MESH)
            return cw, ccw

        local_amax = jnp.float32(0.0)
        for half in range(N_HALF):
            col0 = half * ncol
            cols = pl.ds(col0, ncol)
            partial_into(LO, lax.rem(p + N_DEV - 1, N_DEV), col0)
            partial_into(HI, lax.rem(p + 1, N_DEV), col0 + hcol)
            for j in range(n_strip):
                sl = pl.ds(j * STRIP, STRIP)
                comm_ref[0, :, sl] = pbuf_ref[:, sl]
            for t in range(N_DEV - 1):
                s = t % 2
                r = (t + 1) % 2
                if not (half == 0 and t == 0):
                    pl.semaphore_wait(credit_rs_cw, 1)
                    pl.semaphore_wait(credit_rs_ccw, 1)
                cw, ccw = ring_pair(comm_ref, s, r,
                                    rs_send_cw, rs_recv_cw,
                                    rs_send_ccw, rs_recv_ccw)
                cw.start()
                ccw.start()
                partial_into(LO, lax.rem(p + 2 * N_DEV - 2 - t, N_DEV), col0)
                partial_into(HI, lax.rem(p + 2 + t, N_DEV), col0 + hcol)
                cw.wait()
                ccw.wait()
                if t <= N_DEV - 3:
                    signal(credit_rs_cw, left)
                    signal(credit_rs_ccw, right)
                accum_partial(r, relu=(t == N_DEV - 2))
            rf = (N_DEV - 1) % 2
            local_amax = jnp.maximum(local_amax, jnp.max(comm_ref[rf]))
            park = pltpu.make_async_copy(
                comm_ref.at[rf], out_ref.at[my_rows, cols], out_lo_sem)
            park.start()
            park.wait()
            if half == 0:
                signal(credit_rs_cw, left)
                signal(credit_rs_ccw, right)

        amax_ref[pl.ds(p, 1), :] = jnp.full((1, 128), local_amax, f32)
        for h in range(N_DEV - 1):
            row_s = lax.rem(p + N_DEV - h, N_DEV)
            sc = pltpu.make_async_remote_copy(
                src_ref=amax_ref.at[pl.ds(row_s, 1), :],
                dst_ref=amax_ref.at[pl.ds(row_s, 1), :],
                send_sem=sc_send.at[h], recv_sem=sc_recv.at[h],
                device_id=(right,), device_id_type=pl.DeviceIdType.MESH)
            sc.start()
            sc.wait()
        amax_g = jnp.max(amax_ref[...])
        scale = amax_g / 448.0
        inv = jnp.where(amax_g > 0.0, 448.0 / amax_g, 0.0)

        for half in range(N_HALF):
            col0 = half * ncol
            cols = pl.ds(col0, ncol)
            rb = pltpu.make_async_copy(
                out_ref.at[my_rows, cols], comm_ref.at[0], out_lo_sem)
            rb.start()
            rb.wait()
            for j in range(n_strip):
                sl = pl.ds(j * STRIP, STRIP)
                qcomm_ref[0, :, sl] = (comm_ref[0, :, sl] * inv).astype(f8)
            pend_slot, pend_lo, pend_hi = 0, p, p
            prev_lo = prev_hi = None
            for h in range(N_DEV):
                if h < N_DEV - 1:
                    if not (half == 0 and h == 0):
                        pl.semaphore_wait(credit_ag_cw, 1)
                        pl.semaphore_wait(credit_ag_ccw, 1)
                    cw, ccw = ring_pair(qcomm_ref, h % 2, (h + 1) % 2,
                                        ag_send_cw, ag_recv_cw,
                                        ag_send_ccw, ag_recv_ccw)
                    cw.start()
                    ccw.start()
                if prev_lo is not None:
                    prev_lo.wait()
                    prev_hi.wait()
                comm_ref[1, :, LO] = (
                    qcomm_ref[pend_slot, :, LO].astype(f32) * scale)
                comm_ref[1, :, HI] = (
                    qcomm_ref[pend_slot, :, HI].astype(f32) * scale)
                cp_lo = pltpu.make_async_copy(
                    comm_ref.at[1, :, LO],
                    out_ref.at[pl.ds(pend_lo * chunk, chunk),
                               pl.ds(col0, hcol)],
                    out_lo_sem)
                cp_hi = pltpu.make_async_copy(
                    comm_ref.at[1, :, HI],
                    out_ref.at[pl.ds(pend_hi * chunk, chunk),
                               pl.ds(col0 + hcol, hcol)],
                    out_hi_sem)
                cp_lo.start()
                cp_hi.start()
                prev_lo, prev_hi = cp_lo, cp_hi
                if h < N_DEV - 1:
                    cw.wait()
                    ccw.wait()
                    if h <= N_DEV - 3:
                        signal(credit_ag_cw, left)
                        signal(credit_ag_ccw, right)
                    pend_slot = (h + 1) % 2
                    pend_lo = lax.rem(p + N_DEV - 1 - h, N_DEV)
                    pend_hi = lax.rem(p + 1 + h, N_DEV)
            prev_lo.wait()
            prev_hi.wait()
            if half == 0:
                signal(credit_ag_cw, left)
                signal(credit_ag_ccw, right)

    return pl.pallas_call(
        body,
        out_shape=jax.ShapeDtypeStruct((m, n), f32),
        in_specs=[pl.BlockSpec(memory_space=pl.ANY),
                  pl.BlockSpec(memory_space=pltpu.VMEM)],
        out_specs=pl.BlockSpec(memory_space=pl.ANY),
        scratch_shapes=[
            pltpu.VMEM((2, chunk, ncol), f32),
            pltpu.VMEM((2, chunk, ncol), f8),
            pltpu.VMEM((chunk, ncol), f32),
            pltpu.VMEM((chunk, k_per), f32),
            pltpu.VMEM((N_DEV, 128), f32),
            pltpu.SemaphoreType.DMA((2,)),
            pltpu.SemaphoreType.DMA((2,)),
            pltpu.SemaphoreType.DMA((2,)),
            pltpu.SemaphoreType.DMA((2,)),
            pltpu.SemaphoreType.DMA((2,)),
            pltpu.SemaphoreType.DMA((2,)),
            pltpu.SemaphoreType.DMA((2,)),
            pltpu.SemaphoreType.DMA((2,)),
            pltpu.SemaphoreType.DMA((N_DEV - 1,)),
            pltpu.SemaphoreType.DMA((N_DEV - 1,)),
            pltpu.SemaphoreType.DMA,
            pltpu.SemaphoreType.DMA,
            pltpu.SemaphoreType.DMA,
            pltpu.SemaphoreType.REGULAR,
            pltpu.SemaphoreType.REGULAR,
            pltpu.SemaphoreType.REGULAR,
            pltpu.SemaphoreType.REGULAR,
        ],
        compiler_params=pltpu.CompilerParams(
            collective_id=0, vmem_limit_bytes=46 * 1024 * 1024),
    )(x, w_mat)


# device time: 968266 ns/iter; 1.8781x vs baseline; 1.0553x over previous
import jax
import jax.numpy as jnp
from jax import lax
from jax.experimental import pallas as pl
from jax.experimental.pallas import tpu as pltpu

N_DEV = 8
N_HALF = 2


def kernel(x, w_mat):
    m, k_per = x.shape
    n = w_mat.shape[1]
    chunk = m // N_DEV
    ncol = n // N_HALF
    hcol = ncol // 2
    f32 = jnp.float32
    f8 = jnp.float8_e4m3fn

    def body(x_ref, w_ref, q_ref, scale_ref, park_ref,
             comm_ref, qcomm_ref, pbuf_ref, xbuf_ref, amax_ref,
             rs_send_cw, rs_recv_cw, rs_send_ccw, rs_recv_ccw,
             ag_send_cw, ag_recv_cw, ag_send_ccw, ag_recv_ccw,
             sc_send, sc_recv, out_lo_sem, out_hi_sem, x_sem,
             credit_rs_cw, credit_rs_ccw, credit_ag_cw, credit_ag_ccw):
        p = lax.axis_index("i")
        left = lax.rem(p + N_DEV - 1, N_DEV)
        right = lax.rem(p + 1, N_DEV)
        LO = pl.ds(0, hcol)
        HI = pl.ds(hcol, hcol)

        def signal(sem, target):
            pl.semaphore_signal(sem, inc=1, device_id=(target,),
                                device_id_type=pl.DeviceIdType.MESH)

        barrier = pltpu.get_barrier_semaphore()
        signal(barrier, left)
        signal(barrier, right)
        pl.semaphore_wait(barrier, 2)

        STRIP = 2048
        n_strip = ncol // STRIP

        def load_x(c):
            cp = pltpu.make_async_copy(
                x_ref.at[pl.ds(c * chunk, chunk), :], xbuf_ref, x_sem)
            cp.start()
            cp.wait()

        def partial_into(dst_cols, c, wcol0):
            load_x(c)
            pbuf_ref[:, dst_cols] = jnp.dot(
                xbuf_ref[...], w_ref[:, pl.ds(wcol0, hcol)],
                preferred_element_type=f32)

        def accum_partial(r, relu):
            for j in range(n_strip):
                sl = pl.ds(j * STRIP, STRIP)
                acc = comm_ref[r, :, sl] + pbuf_ref[:, sl]
                if relu:
                    acc = jnp.maximum(acc, 0.0)
                comm_ref[r, :, sl] = acc

        def ring_pair(buf_ref, s, r, send_cw, recv_cw, send_ccw, recv_ccw):
            cw = pltpu.make_async_remote_copy(
                src_ref=buf_ref.at[s, :, LO], dst_ref=buf_ref.at[r, :, LO],
                send_sem=send_cw.at[s], recv_sem=recv_cw.at[r],
                device_id=(right,), device_id_type=pl.DeviceIdType.MESH)
            ccw = pltpu.make_async_remote_copy(
                src_ref=buf_ref.at[s, :, HI], dst_ref=buf_ref.at[r, :, HI],
                send_sem=send_ccw.at[s], recv_sem=recv_ccw.at[r],
                device_id=(left,), device_id_type=pl.DeviceIdType.MESH)
            return cw, ccw

        local_amax = jnp.float32(0.0)
        for half in range(N_HALF):
            col0 = half * ncol
            partial_into(LO, lax.rem(p + N_DEV - 1, N_DEV), col0)
            partial_into(HI, lax.rem(p + 1, N_DEV), col0 + hcol)
            for j in range(n_strip):
                sl = pl.ds(j * STRIP, STRIP)
                comm_ref[0, :, sl] = pbuf_ref[:, sl]
            for t in range(N_DEV - 1):
                s = t % 2
                r = (t + 1) % 2
                if not (half == 0 and t == 0):
                    pl.semaphore_wait(credit_rs_cw, 1)
                    pl.semaphore_wait(credit_rs_ccw, 1)
                cw, ccw = ring_pair(comm_ref, s, r,
                                    rs_send_cw, rs_recv_cw,
                                    rs_send_ccw, rs_recv_ccw)
                cw.start()
                ccw.start()
                partial_into(LO, lax.rem(p + 2 * N_DEV - 2 - t, N_DEV), col0)
                partial_into(HI, lax.rem(p + 2 + t, N_DEV), col0 + hcol)
                cw.wait()
                ccw.wait()
                if t <= N_DEV - 3:
                    signal(credit_rs_cw, left)
                    signal(credit_rs_ccw, right)
                accum_partial(r, relu=(t == N_DEV - 2))
            local_amax = jnp.maximum(local_amax, jnp.max(comm_ref[1]))
            if half == 0:
                park = pltpu.make_async_copy(
                    comm_ref.at[1], park_ref, out_lo_sem)
                park.start()
                park.wait()
                signal(credit_rs_cw, left)
                signal(credit_rs_ccw, right)

        amax_ref[pl.ds(p, 1), :] = jnp.full((1, 128), local_amax, f32)
        sends = []
        for k in range(1, N_DEV):
            sc = pltpu.make_async_remote_copy(
                src_ref=amax_ref.at[pl.ds(p, 1), :],
                dst_ref=amax_ref.at[pl.ds(p, 1), :],
                send_sem=sc_send.at[k - 1], recv_sem=sc_recv.at[k - 1],
                device_id=(lax.rem(p + k, N_DEV),),
                device_id_type=pl.DeviceIdType.MESH)
            sc.start()
            sends.append(sc)
        for k in range(1, N_DEV):
            row = pl.ds(lax.rem(p + N_DEV - k, N_DEV), 1)
            recv = pltpu.make_async_remote_copy(
                src_ref=amax_ref.at[row, :], dst_ref=amax_ref.at[row, :],
                send_sem=sc_send.at[k - 1], recv_sem=sc_recv.at[k - 1],
                device_id=(p,), device_id_type=pl.DeviceIdType.MESH)
            recv.wait_recv()
        for sc in sends:
            sc.wait_send()
        amax_g = jnp.max(amax_ref[...])
        scale = amax_g / 448.0
        inv = jnp.where(amax_g > 0.0, 448.0 / amax_g, 0.0)
        scale_ref[...] = jnp.full((N_DEV, 128), scale, f32)

        for idx, half in enumerate((1, 0)):
            col0 = half * ncol
            if half == 0:
                rb = pltpu.make_async_copy(
                    park_ref, comm_ref.at[0], out_lo_sem)
                rb.start()
                rb.wait()
                src_slot = 0
            else:
                src_slot = 1
            for j in range(n_strip):
                sl = pl.ds(j * STRIP, STRIP)
                qcomm_ref[0, :, sl] = (
                    comm_ref[src_slot, :, sl] * inv).astype(f8)
            pend_slot, pend_lo, pend_hi = 0, p, p
            for h in range(N_DEV):
                if h < N_DEV - 1:
                    if not (idx == 0 and h == 0):
                        pl.semaphore_wait(credit_ag_cw, 1)
                        pl.semaphore_wait(credit_ag_ccw, 1)
                    cw, ccw = ring_pair(qcomm_ref, h % 2, (h + 1) % 2,
                                        ag_send_cw, ag_recv_cw,
                                        ag_send_ccw, ag_recv_ccw)
                    cw.start()
                    ccw.start()
                cp_lo = pltpu.make_async_copy(
                    qcomm_ref.at[pend_slot, :, LO],
                    q_ref.at[pl.ds(pend_lo * chunk, chunk),
                             pl.ds(col0, hcol)],
                    out_lo_sem)
                cp_hi = pltpu.make_async_copy(
                    qcomm_ref.at[pend_slot, :, HI],
                    q_ref.at[pl.ds(pend_hi * chunk, chunk),
                             pl.ds(col0 + hcol, hcol)],
                    out_hi_sem)
                cp_lo.start()
                cp_hi.start()
                cp_lo.wait()
                cp_hi.wait()
                if h < N_DEV - 1:
                    cw.wait()
                    ccw.wait()
                    if h <= N_DEV - 3:
                        signal(credit_ag_cw, left)
                        signal(credit_ag_ccw, right)
                    pend_slot = (h + 1) % 2
                    pend_lo = lax.rem(p + N_DEV - 1 - h, N_DEV)
                    pend_hi = lax.rem(p + 1 + h, N_DEV)
            if idx == 0:
                signal(credit_ag_cw, left)
                signal(credit_ag_ccw, right)

    q, scl, _park = pl.pallas_call(
        body,
        out_shape=[
            jax.ShapeDtypeStruct((m, n), f8),
            jax.ShapeDtypeStruct((N_DEV, 128), f32),
            jax.ShapeDtypeStruct((chunk, ncol), f32),
        ],
        in_specs=[pl.BlockSpec(memory_space=pl.ANY),
                  pl.BlockSpec(memory_space=pltpu.VMEM)],
        out_specs=[pl.BlockSpec(memory_space=pl.ANY),
                   pl.BlockSpec(memory_space=pltpu.VMEM),
                   pl.BlockSpec(memory_space=pl.ANY)],
        scratch_shapes=[
            pltpu.VMEM((2, chunk, ncol), f32),
            pltpu.VMEM((2, chunk, ncol), f8),
            pltpu.VMEM((chunk, ncol), f32),
            pltpu.VMEM((chunk, k_per), f32),
            pltpu.VMEM((N_DEV, 128), f32),
            pltpu.SemaphoreType.DMA((2,)),
            pltpu.SemaphoreType.DMA((2,)),
            pltpu.SemaphoreType.DMA((2,)),
            pltpu.SemaphoreType.DMA((2,)),
            pltpu.SemaphoreType.DMA((2,)),
            pltpu.SemaphoreType.DMA((2,)),
            pltpu.SemaphoreType.DMA((2,)),
            pltpu.SemaphoreType.DMA((2,)),
            pltpu.SemaphoreType.DMA((N_DEV - 1,)),
            pltpu.SemaphoreType.DMA((N_DEV - 1,)),
            pltpu.SemaphoreType.DMA,
            pltpu.SemaphoreType.DMA,
            pltpu.SemaphoreType.DMA,
            pltpu.SemaphoreType.REGULAR,
            pltpu.SemaphoreType.REGULAR,
            pltpu.SemaphoreType.REGULAR,
            pltpu.SemaphoreType.REGULAR,
        ],
        compiler_params=pltpu.CompilerParams(
            collective_id=0, vmem_limit_bytes=46 * 1024 * 1024),
    )(x, w_mat)
    return q.astype(f32) * scl[0, 0]


# device time: 965698 ns/iter; 1.8831x vs baseline; 1.0027x over previous
import jax
import jax.numpy as jnp
from jax import lax
from jax.experimental import pallas as pl
from jax.experimental.pallas import tpu as pltpu

N_DEV = 8
N_HALF = 2


def kernel(x, w_mat):
    m, k_per = x.shape
    n = w_mat.shape[1]
    chunk = m // N_DEV
    ncol = n // N_HALF
    hcol = ncol // 2
    f32 = jnp.float32
    f8 = jnp.float8_e4m3fn

    def body(x_ref, w_ref, q_ref, scale_ref, park_ref,
             comm_ref, qcomm_ref, pbuf_ref, xbuf_ref, amax_ref,
             rs_send_cw, rs_recv_cw, rs_send_ccw, rs_recv_ccw,
             ag_send_cw, ag_recv_cw, ag_send_ccw, ag_recv_ccw,
             sc_send, sc_recv, out_lo_sem, out_hi_sem, x_sem,
             credit_rs_cw, credit_rs_ccw, credit_ag_cw, credit_ag_ccw):
        p = lax.axis_index("i")
        left = lax.rem(p + N_DEV - 1, N_DEV)
        right = lax.rem(p + 1, N_DEV)
        LO = pl.ds(0, hcol)
        HI = pl.ds(hcol, hcol)

        def signal(sem, target):
            pl.semaphore_signal(sem, inc=1, device_id=(target,),
                                device_id_type=pl.DeviceIdType.MESH)

        barrier = pltpu.get_barrier_semaphore()
        signal(barrier, left)
        signal(barrier, right)
        pl.semaphore_wait(barrier, 2)

        STRIP = 2048
        n_strip = ncol // STRIP

        def load_x(c):
            cp = pltpu.make_async_copy(
                x_ref.at[pl.ds(c * chunk, chunk), :], xbuf_ref, x_sem)
            cp.start()
            cp.wait()

        def partial_into(dst_cols, c, wcol0):
            load_x(c)
            pbuf_ref[:, dst_cols] = jnp.dot(
                xbuf_ref[...], w_ref[:, pl.ds(wcol0, hcol)],
                preferred_element_type=f32)

        def accum_quarter(r, qidx, relu):
            sl = pl.ds(qidx * (hcol // 2), hcol // 2)
            acc = comm_ref[r, :, sl] + pbuf_ref[:, sl]
            if relu:
                acc = jnp.maximum(acc, 0.0)
            comm_ref[r, :, sl] = acc

        def rs_quad(s, r):
            qcol = hcol // 2
            subs = []
            for sub in range(4):
                sl = pl.ds(sub * qcol, qcol)
                cwdir = sub < 2
                subs.append(pltpu.make_async_remote_copy(
                    src_ref=comm_ref.at[s, :, sl],
                    dst_ref=comm_ref.at[r, :, sl],
                    send_sem=(rs_send_cw if cwdir else rs_send_ccw).at[s, sub % 2],
                    recv_sem=(rs_recv_cw if cwdir else rs_recv_ccw).at[r, sub % 2],
                    device_id=(right if cwdir else left,),
                    device_id_type=pl.DeviceIdType.MESH))
            return subs

        def ring_pair(buf_ref, s, r, send_cw, recv_cw, send_ccw, recv_ccw):
            cw = pltpu.make_async_remote_copy(
                src_ref=buf_ref.at[s, :, LO], dst_ref=buf_ref.at[r, :, LO],
                send_sem=send_cw.at[s], recv_sem=recv_cw.at[r],
                device_id=(right,), device_id_type=pl.DeviceIdType.MESH)
            ccw = pltpu.make_async_remote_copy(
                src_ref=buf_ref.at[s, :, HI], dst_ref=buf_ref.at[r, :, HI],
                send_sem=send_ccw.at[s], recv_sem=recv_ccw.at[r],
                device_id=(left,), device_id_type=pl.DeviceIdType.MESH)
            return cw, ccw

        local_amax = jnp.float32(0.0)
        for half in range(N_HALF):
            col0 = half * ncol
            partial_into(LO, lax.rem(p + N_DEV - 1, N_DEV), col0)
            partial_into(HI, lax.rem(p + 1, N_DEV), col0 + hcol)
            for j in range(n_strip):
                sl = pl.ds(j * STRIP, STRIP)
                comm_ref[0, :, sl] = pbuf_ref[:, sl]
            for t in range(N_DEV - 1):
                s = t % 2
                r = (t + 1) % 2
                if not (half == 0 and t == 0):
                    pl.semaphore_wait(credit_rs_cw, 1)
                    pl.semaphore_wait(credit_rs_ccw, 1)
                sub = rs_quad(s, r)
                for d in sub:
                    d.start()
                partial_into(LO, lax.rem(p + 2 * N_DEV - 2 - t, N_DEV), col0)
                partial_into(HI, lax.rem(p + 2 + t, N_DEV), col0 + hcol)
                relu = t == N_DEV - 2
                sub[0].wait()
                accum_quarter(r, 0, relu)
                sub[2].wait()
                accum_quarter(r, 2, relu)
                sub[1].wait()
                sub[3].wait()
                if t <= N_DEV - 3:
                    signal(credit_rs_cw, left)
                    signal(credit_rs_ccw, right)
                accum_quarter(r, 1, relu)
                accum_quarter(r, 3, relu)
            local_amax = jnp.maximum(local_amax, jnp.max(comm_ref[1]))
            if half == 0:
                park = pltpu.make_async_copy(
                    comm_ref.at[1], park_ref, out_lo_sem)
                park.start()
                park.wait()
                signal(credit_rs_cw, left)
                signal(credit_rs_ccw, right)

        amax_ref[pl.ds(p, 1), :] = jnp.full((1, 128), local_amax, f32)
        sends = []
        for k in range(1, N_DEV):
            sc = pltpu.make_async_remote_copy(
                src_ref=amax_ref.at[pl.ds(p, 1), :],
                dst_ref=amax_ref.at[pl.ds(p, 1), :],
                send_sem=sc_send.at[k - 1], recv_sem=sc_recv.at[k - 1],
                device_id=(lax.rem(p + k, N_DEV),),
                device_id_type=pl.DeviceIdType.MESH)
            sc.start()
            sends.append(sc)
        for k in range(1, N_DEV):
            row = pl.ds(lax.rem(p + N_DEV - k, N_DEV), 1)
            recv = pltpu.make_async_remote_copy(
                src_ref=amax_ref.at[row, :], dst_ref=amax_ref.at[row, :],
                send_sem=sc_send.at[k - 1], recv_sem=sc_recv.at[k - 1],
                device_id=(p,), device_id_type=pl.DeviceIdType.MESH)
            recv.wait_recv()
        for sc in sends:
            sc.wait_send()
        amax_g = jnp.max(amax_ref[...])
        scale = amax_g / 448.0
        inv = jnp.where(amax_g > 0.0, 448.0 / amax_g, 0.0)
        scale_ref[...] = jnp.full((N_DEV, 128), scale, f32)

        for idx, half in enumerate((1, 0)):
            col0 = half * ncol
            if half == 0:
                rb = pltpu.make_async_copy(
                    park_ref, comm_ref.at[0], out_lo_sem)
                rb.start()
                rb.wait()
                src_slot = 0
            else:
                src_slot = 1
            for j in range(n_strip):
                sl = pl.ds(j * STRIP, STRIP)
                qcomm_ref[0, :, sl] = (
                    comm_ref[src_slot, :, sl] * inv).astype(f8)
            pend_slot, pend_lo, pend_hi = 0, p, p
            for h in range(N_DEV):
                if h < N_DEV - 1:
                    if not (idx == 0 and h == 0):
                        pl.semaphore_wait(credit_ag_cw, 1)
                        pl.semaphore_wait(credit_ag_ccw, 1)
                    cw, ccw = ring_pair(qcomm_ref, h % 2, (h + 1) % 2,
                                        ag_send_cw, ag_recv_cw,
                                        ag_send_ccw, ag_recv_ccw)
                    cw.start()
                    ccw.start()
                cp_lo = pltpu.make_async_copy(
                    qcomm_ref.at[pend_slot, :, LO],
                    q_ref.at[pl.ds(pend_lo * chunk, chunk),
                             pl.ds(col0, hcol)],
                    out_lo_sem)
                cp_hi = pltpu.make_async_copy(
                    qcomm_ref.at[pend_slot, :, HI],
                    q_ref.at[pl.ds(pend_hi * chunk, chunk),
                             pl.ds(col0 + hcol, hcol)],
                    out_hi_sem)
                cp_lo.start()
                cp_hi.start()
                cp_lo.wait()
                cp_hi.wait()
                if h < N_DEV - 1:
                    cw.wait()
                    ccw.wait()
                    if h <= N_DEV - 3:
                        signal(credit_ag_cw, left)
                        signal(credit_ag_ccw, right)
                    pend_slot = (h + 1) % 2
                    pend_lo = lax.rem(p + N_DEV - 1 - h, N_DEV)
                    pend_hi = lax.rem(p + 1 + h, N_DEV)
            if idx == 0:
                signal(credit_ag_cw, left)
                signal(credit_ag_ccw, right)

    q, scl, _park = pl.pallas_call(
        body,
        out_shape=[
            jax.ShapeDtypeStruct((m, n), f8),
            jax.ShapeDtypeStruct((N_DEV, 128), f32),
            jax.ShapeDtypeStruct((chunk, ncol), f32),
        ],
        in_specs=[pl.BlockSpec(memory_space=pl.ANY),
                  pl.BlockSpec(memory_space=pltpu.VMEM)],
        out_specs=[pl.BlockSpec(memory_space=pl.ANY),
                   pl.BlockSpec(memory_space=pltpu.VMEM),
                   pl.BlockSpec(memory_space=pl.ANY)],
        scratch_shapes=[
            pltpu.VMEM((2, chunk, ncol), f32),
            pltpu.VMEM((2, chunk, ncol), f8),
            pltpu.VMEM((chunk, ncol), f32),
            pltpu.VMEM((chunk, k_per), f32),
            pltpu.VMEM((N_DEV, 128), f32),
            pltpu.SemaphoreType.DMA((2, 2)),
            pltpu.SemaphoreType.DMA((2, 2)),
            pltpu.SemaphoreType.DMA((2, 2)),
            pltpu.SemaphoreType.DMA((2, 2)),
            pltpu.SemaphoreType.DMA((2,)),
            pltpu.SemaphoreType.DMA((2,)),
            pltpu.SemaphoreType.DMA((2,)),
            pltpu.SemaphoreType.DMA((2,)),
            pltpu.SemaphoreType.DMA((N_DEV - 1,)),
            pltpu.SemaphoreType.DMA((N_DEV - 1,)),
            pltpu.SemaphoreType.DMA,
            pltpu.SemaphoreType.DMA,
            pltpu.SemaphoreType.DMA,
            pltpu.SemaphoreType.REGULAR,
            pltpu.SemaphoreType.REGULAR,
            pltpu.SemaphoreType.REGULAR,
            pltpu.SemaphoreType.REGULAR,
        ],
        compiler_params=pltpu.CompilerParams(
            collective_id=0, vmem_limit_bytes=46 * 1024 * 1024),
    )(x, w_mat)
    return q.astype(f32) * scl[0, 0]
